# baseline (speedup 1.0000x reference)
import functools

import jax
import jax.numpy as jnp
from jax import lax
from jax.experimental import pallas as pl
from jax.experimental.pallas import tpu as pltpu
from jax.experimental.pallas import tpu_sc as plsc

EMB_DIM = 64
NUM_CORES = 2
NUM_SUBCORES = 16
NUM_WORKERS = NUM_CORES * NUM_SUBCORES
CHUNK = 128


@functools.partial(jax.jit, static_argnums=(2, 3))
def _gather_rows(x3, weights, b_per_w, nchunk):
    B = NUM_WORKERS * b_per_w
    mesh = plsc.VectorSubcoreMesh(core_axis_name="c", subcore_axis_name="s")

    @functools.partial(
        pl.kernel,
        mesh=mesh,
        out_type=jax.ShapeDtypeStruct((B, EMB_DIM), jnp.float32),
        compiler_params=pltpu.CompilerParams(use_tc_tiling_on_sc=False),
        scratch_types=[
            pltpu.VMEM((nchunk, CHUNK), jnp.int32),
            pltpu.VMEM((CHUNK, EMB_DIM), jnp.float32),
            pltpu.VMEM((CHUNK, EMB_DIM), jnp.float32),
            pltpu.SemaphoreType.DMA,
            pltpu.SemaphoreType.DMA,
        ],
    )
    def k(x_hbm, w_hbm, out_hbm, idx_v, buf0, buf1, sem0, sem1):
        wid = lax.axis_index("s") * NUM_CORES + lax.axis_index("c")
        base = wid * b_per_w
        pltpu.sync_copy(x_hbm.at[wid], idx_v)
        bufs = (buf0, buf1)
        sems = (sem0, sem1)
        pltpu.make_async_copy(w_hbm.at[idx_v.at[0]], buf0, sem0).start()
        pltpu.make_async_copy(w_hbm.at[idx_v.at[1]], buf1, sem1).start()

        def body(jj, carry):
            for b in range(2):
                j = jj * 2 + b
                pltpu.make_async_copy(w_hbm.at[idx_v.at[j]], bufs[b], sems[b]).wait()
                pltpu.sync_copy(
                    bufs[b], out_hbm.at[pl.ds(base + j * CHUNK, CHUNK)]
                )

                @pl.when(jj < nchunk // 2 - 1)
                def _():
                    pltpu.make_async_copy(
                        w_hbm.at[idx_v.at[j + 2]], bufs[b], sems[b]
                    ).start()

            return carry

        lax.fori_loop(0, nchunk // 2, body, 0)

    return k(x3, weights)


def kernel(x, weights):
    orig_shape = x.shape
    B = x.size
    assert B % (NUM_WORKERS * CHUNK) == 0
    b_per_w = B // NUM_WORKERS
    nchunk = b_per_w // CHUNK
    x3 = x.reshape(NUM_WORKERS, nchunk, CHUNK).astype(jnp.int32)
    rows = _gather_rows(x3, weights, b_per_w, nchunk)
    return rows.reshape(*orig_shape, EMB_DIM)


# speedup vs baseline: 1.0131x; 1.0131x over previous
import functools

import jax
import jax.numpy as jnp
from jax import lax
from jax.experimental import pallas as pl
from jax.experimental.pallas import tpu as pltpu
from jax.experimental.pallas import tpu_sc as plsc

EMB_DIM = 64
NUM_CORES = 2
NUM_SUBCORES = 16
NUM_WORKERS = NUM_CORES * NUM_SUBCORES
CHUNK = 128
N_BUF = 8
K_AHEAD = 6


@functools.partial(jax.jit, static_argnums=(2, 3))
def _gather_rows(x3, weights, b_per_w, nchunk):
    B = NUM_WORKERS * b_per_w
    nblk = nchunk // N_BUF
    mesh = plsc.VectorSubcoreMesh(core_axis_name="c", subcore_axis_name="s")

    @functools.partial(
        pl.kernel,
        mesh=mesh,
        out_type=jax.ShapeDtypeStruct((B, EMB_DIM), jnp.float32),
        compiler_params=pltpu.CompilerParams(use_tc_tiling_on_sc=False),
        scratch_types=(
            [pltpu.VMEM((nchunk, CHUNK), jnp.int32)]
            + [pltpu.VMEM((CHUNK, EMB_DIM), jnp.float32)] * N_BUF
            + [pltpu.SemaphoreType.DMA] * (2 * N_BUF)
        ),
    )
    def k(x_hbm, w_hbm, out_hbm, idx_v, *rest):
        bufs = rest[:N_BUF]
        gsems = rest[N_BUF : 2 * N_BUF]
        osems = rest[2 * N_BUF :]
        wid = lax.axis_index("s") * NUM_CORES + lax.axis_index("c")
        base = wid * b_per_w
        pltpu.sync_copy(x_hbm.at[wid], idx_v)

        def gather_start(j, b):
            pltpu.make_async_copy(w_hbm.at[idx_v.at[j]], bufs[b], gsems[b]).start()

        def gather_wait(b):
            pltpu.make_async_copy(w_hbm.at[idx_v.at[0]], bufs[b], gsems[b]).wait()

        def store_start(j, b):
            pltpu.make_async_copy(
                bufs[b], out_hbm.at[pl.ds(base + j * CHUNK, CHUNK)], osems[b]
            ).start()

        def store_wait(b):
            pltpu.make_async_copy(
                bufs[b], out_hbm.at[pl.ds(base, CHUNK)], osems[b]
            ).wait()

        def block(jj, first=False, last=False):
            for b in range(N_BUF):
                j = jj * N_BUF + b
                gather_wait(b)
                store_start(j, b)
                bk = (b + K_AHEAD) % N_BUF
                if last and b >= N_BUF - K_AHEAD:
                    continue
                if not (first and b < N_BUF - K_AHEAD):
                    store_wait(bk)
                gather_start(j + K_AHEAD, bk)

        for j in range(K_AHEAD):
            gather_start(j, j)
        block(0, first=True)
        lax.fori_loop(1, nblk - 1, lambda jj, c: (block(jj), c)[1], 0)
        block(nblk - 1, last=True)
        for b in range(N_BUF):
            store_wait(b)

    return k(x3, weights)


def kernel(x, weights):
    orig_shape = x.shape
    B = x.size
    assert B % (NUM_WORKERS * CHUNK) == 0
    b_per_w = B // NUM_WORKERS
    nchunk = b_per_w // CHUNK
    x3 = x.reshape(NUM_WORKERS, nchunk, CHUNK).astype(jnp.int32)
    rows = _gather_rows(x3, weights, b_per_w, nchunk)
    return rows.reshape(*orig_shape, EMB_DIM)
